# baseline (speedup 1.0000x reference)
import math

import jax
import jax.numpy as jnp
import numpy as np
from jax.experimental import pallas as pl

_PATCH = (2, 2)
_IMAGE_DIM = (64, 64)
_N_EMBED = 8192
_NEIGH = 256
_VARIANCE = -(_NEIGH / (2.0 * math.log(0.1)))
_M_TILE = 256


def _patchify(image):
    pH, pW = _PATCH
    N, C, H, W = image.shape
    x = image.reshape(N, C, H // pH, pH, W // pW, pW)
    x = jnp.transpose(x, (0, 2, 4, 1, 3, 5))
    return x.reshape(N, (H // pH) * (W // pW), C * pH * pW)


def _unpatchify(patches, image_dim, patch_dim):
    H, W = image_dim
    pH, pW = patch_dim
    N, Seq, D = patches.shape
    C = D // (pH * pW)
    x = patches.reshape(N, H // pH, W // pW, C, pH, pW)
    x = jnp.transpose(x, (0, 3, 1, 4, 2, 5))
    return x.reshape(N, C, H, W)


def _vq_body(f_ref, cb_ref, out_ref):
    f = f_ref[...]
    cb = cb_ref[...]
    a = jnp.sum(f * f, axis=1, keepdims=True)
    b = jnp.sum(cb * cb, axis=1)[None, :]
    c = jax.lax.dot_general(f, cb, (((1,), (1,)), ((), ())),
                            preferred_element_type=jnp.float32)
    d2 = (a + b) - 2.0 * c
    m = jnp.min(d2, axis=1, keepdims=True)
    idx = jax.lax.broadcasted_iota(jnp.int32, d2.shape, 1)
    bmu = jnp.min(jnp.where(d2 == m, idx, _N_EMBED), axis=1)
    delta = (idx - bmu[:, None]).astype(jnp.float32)
    scale = jnp.exp(-(delta * delta) / (2.0 * _VARIANCE))
    out_ref[...] = jax.lax.dot_general(
        scale, cb, (((1,), (0,)), ((), ())),
        preferred_element_type=jnp.float32)


def kernel(x, codebook):
    patches = _patchify(x)
    N, Seq, D = patches.shape
    flat = patches.reshape(N * Seq, D)
    M = N * Seq
    quant = pl.pallas_call(
        _vq_body,
        grid=(M // _M_TILE,),
        in_specs=[
            pl.BlockSpec((_M_TILE, D), lambda i: (i, 0)),
            pl.BlockSpec((_N_EMBED, D), lambda i: (0, 0)),
        ],
        out_specs=pl.BlockSpec((_M_TILE, D), lambda i: (i, 0)),
        out_shape=jax.ShapeDtypeStruct((M, D), jnp.float32),
    )(flat, codebook)
    return _unpatchify(quant.reshape(N, Seq, D), _IMAGE_DIM, _PATCH)
